# baseline (speedup 1.0000x reference)
import functools

import jax
import jax.numpy as jnp
from jax import lax
from jax.experimental import pallas as pl
from jax.experimental.pallas import tpu as pltpu
from jax.experimental.pallas import tpu_sc as plsc

N_NODES = 10000
N_NODES_PAD = 10240
EDGES = 320000
NC, NS = 2, 16
PER_SUBCORE = EDGES // NS
NSTREAM = 5
STREAM = PER_SUBCORE // NSTREAM
SLICE = N_NODES_PAD // NS
TAIL = N_NODES - (NS - 1) * SLICE

_mesh = plsc.VectorSubcoreMesh(
    core_axis_name="c", subcore_axis_name="s", num_cores=NC, num_subcores=NS
)


@functools.partial(
    pl.kernel,
    out_type=(
        jax.ShapeDtypeStruct((N_NODES,), jnp.int32),
        jax.ShapeDtypeStruct((N_NODES,), jnp.int32),
    ),
    mesh=_mesh,
    scratch_types=[
        [pltpu.VMEM((STREAM,), jnp.int32) for _ in range(NSTREAM)],
        pltpu.VMEM((STREAM,), jnp.int32),
        pltpu.VMEM((SLICE,), jnp.int32),
        pltpu.VMEM_SHARED((N_NODES_PAD,), jnp.int32),
        pltpu.SemaphoreType.DMA,
        pltpu.SemaphoreType.DMA,
    ],
    compiler_params=pltpu.CompilerParams(use_tc_tiling_on_sc=False),
)
def _degree_sc(e0_hbm, e1_hbm, out0_hbm, out1_hbm,
               idx_v, ones_v, zero_v, hist_s, sem, sem2):
    c = lax.axis_index("c")
    s = lax.axis_index("s")

    def fill_ones(i, carry):
        ones_v[pl.ds(pl.multiple_of(i * 16, 16), 16)] = jnp.full((16,), 1, jnp.int32)
        return carry

    def run(src_hbm, dst_hbm):
        idx_cps = [
            pltpu.async_copy(
                src_hbm.at[pl.ds(s * PER_SUBCORE + k * STREAM, STREAM)],
                idx_v[k],
                sem,
            )
            for k in range(NSTREAM)
        ]
        lax.fori_loop(0, STREAM // 16, fill_ones, 0)
        for k in range(SLICE // 16):
            zero_v[pl.ds(k * 16, 16)] = jnp.zeros((16,), jnp.int32)
        pltpu.sync_copy(zero_v, hist_s.at[pl.ds(s * SLICE, SLICE)])
        plsc.subcore_barrier()

        adds = []
        for k in range(NSTREAM):
            idx_cps[k].wait()
            adds.append(
                pltpu.async_copy(ones_v, hist_s.at[idx_v[k]], sem2, add=True)
            )
        for cp in adds:
            cp.wait()
        plsc.subcore_barrier()

        @pl.when(s < NS - 1)
        def _():
            pltpu.sync_copy(hist_s.at[pl.ds(s * SLICE, SLICE)],
                            dst_hbm.at[pl.ds(s * SLICE, SLICE)])

        @pl.when(s == NS - 1)
        def _():
            pltpu.sync_copy(hist_s.at[pl.ds((NS - 1) * SLICE, TAIL)],
                            dst_hbm.at[pl.ds((NS - 1) * SLICE, TAIL)])

    @pl.when(c == 0)
    def _core0():
        run(e0_hbm, out0_hbm)

    @pl.when(c == 1)
    def _core1():
        run(e1_hbm, out1_hbm)


def _detile_body(e_ref, o0_ref, o1_ref):
    o0_ref[...] = e_ref[0]
    o1_ref[...] = e_ref[1]


_detile = pl.pallas_call(
    _detile_body,
    grid=(20,),
    in_specs=[pl.BlockSpec((2, 16384), lambda i: (0, i))],
    out_specs=[
        pl.BlockSpec((16384,), lambda i: (i,)),
        pl.BlockSpec((16384,), lambda i: (i,)),
    ],
    out_shape=(
        jax.ShapeDtypeStruct((EDGES,), jnp.int32),
        jax.ShapeDtypeStruct((EDGES,), jnp.int32),
    ),
)


def _copy_body(x_ref, o_ref):
    o_ref[...] = x_ref[...]


_copy_x = pl.pallas_call(
    _copy_body,
    grid=(5,),
    in_specs=[pl.BlockSpec((2000, 128), lambda i: (i, 0))],
    out_specs=pl.BlockSpec((2000, 128), lambda i: (i, 0)),
    out_shape=jax.ShapeDtypeStruct((10000, 128), jnp.float32),
)


def kernel(x, edge_index):
    ei = edge_index.astype(jnp.int32)
    e0, e1 = _detile(ei)
    out_degree, in_degree = _degree_sc(e0, e1)
    x_out = _copy_x(x)
    out_dtype = jax.dtypes.canonicalize_dtype(jnp.int64)
    return x_out, in_degree.astype(out_dtype), out_degree.astype(out_dtype)


# speedup vs baseline: 1.0508x; 1.0508x over previous
import functools

import jax
import jax.numpy as jnp
from jax import lax
from jax.experimental import pallas as pl
from jax.experimental.pallas import tpu as pltpu
from jax.experimental.pallas import tpu_sc as plsc

N_NODES = 10000
N_NODES_PAD = 10240
EDGES = 320000
NC, NS = 2, 16
PER_SUBCORE = EDGES // NS
NSTREAM = 5
STREAM = PER_SUBCORE // NSTREAM
SLICE = N_NODES_PAD // NS
TAIL = N_NODES - (NS - 1) * SLICE

_mesh = plsc.VectorSubcoreMesh(
    core_axis_name="c", subcore_axis_name="s", num_cores=NC, num_subcores=NS
)


@functools.partial(
    pl.kernel,
    out_type=(
        jax.ShapeDtypeStruct((N_NODES,), jnp.int32),
        jax.ShapeDtypeStruct((N_NODES,), jnp.int32),
    ),
    mesh=_mesh,
    scratch_types=[
        [pltpu.VMEM((STREAM,), jnp.int32) for _ in range(NSTREAM)],
        pltpu.VMEM((STREAM,), jnp.int32),
        pltpu.VMEM_SHARED((N_NODES_PAD,), jnp.int32),
        pltpu.SemaphoreType.DMA,
        pltpu.SemaphoreType.DMA,
    ],
    compiler_params=pltpu.CompilerParams(use_tc_tiling_on_sc=False),
)
def _degree_sc(edge_hbm, ones_hbm, zeros_hbm, out0_hbm, out1_hbm,
               idx_v, ones_v, hist_s, sem, sem2):
    c = lax.axis_index("c")
    s = lax.axis_index("s")

    idx_cps = [
        pltpu.async_copy(
            edge_hbm.at[c, pl.ds(s * PER_SUBCORE + k * STREAM, STREAM)],
            idx_v[k],
            sem,
        )
        for k in range(NSTREAM)
    ]
    ones_cp = pltpu.async_copy(ones_hbm, ones_v, sem2)
    pltpu.sync_copy(zeros_hbm, hist_s.at[pl.ds(s * SLICE, SLICE)])
    ones_cp.wait()
    plsc.subcore_barrier()

    adds = []
    for k in range(NSTREAM):
        idx_cps[k].wait()
        adds.append(
            pltpu.async_copy(ones_v, hist_s.at[idx_v[k]], sem2, add=True)
        )
    for cp in adds:
        cp.wait()
    plsc.subcore_barrier()

    @pl.when(s < NS - 1)
    def _full_slice():
        for dst in (out0_hbm, out1_hbm):
            @pl.when((c == 0) == (dst is out0_hbm))
            def _():
                pltpu.sync_copy(hist_s.at[pl.ds(s * SLICE, SLICE)],
                                dst.at[pl.ds(s * SLICE, SLICE)])

    @pl.when(s == NS - 1)
    def _tail_slice():
        for dst in (out0_hbm, out1_hbm):
            @pl.when((c == 0) == (dst is out0_hbm))
            def _():
                pltpu.sync_copy(hist_s.at[pl.ds((NS - 1) * SLICE, TAIL)],
                                dst.at[pl.ds((NS - 1) * SLICE, TAIL)])


def _copy_body(x_ref, o_ref):
    o_ref[...] = x_ref[...]


_copy_x = pl.pallas_call(
    _copy_body,
    grid=(5,),
    in_specs=[pl.BlockSpec((2000, 128), lambda i: (i, 0))],
    out_specs=pl.BlockSpec((2000, 128), lambda i: (i, 0)),
    out_shape=jax.ShapeDtypeStruct((10000, 128), jnp.float32),
)


def kernel(x, edge_index):
    ei = edge_index.astype(jnp.int32)
    ones_c = jnp.ones((STREAM,), jnp.int32)
    zeros_c = jnp.zeros((SLICE,), jnp.int32)
    out_degree, in_degree = _degree_sc(ei, ones_c, zeros_c)
    x_out = _copy_x(x)
    out_dtype = jax.dtypes.canonicalize_dtype(jnp.int64)
    return x_out, in_degree.astype(out_dtype), out_degree.astype(out_dtype)


# speedup vs baseline: 1.2241x; 1.1650x over previous
import functools

import jax
import jax.numpy as jnp
from jax import lax
from jax.experimental import pallas as pl
from jax.experimental.pallas import tpu as pltpu
from jax.experimental.pallas import tpu_sc as plsc

N_NODES = 10000
N_NODES_PAD = 10240
EDGES = 320000
NC, NS = 2, 16
PER_SUBCORE = EDGES // NS
NSTREAM = 5
STREAM = PER_SUBCORE // NSTREAM
SLICE = N_NODES_PAD // NS
TAIL = N_NODES - (NS - 1) * SLICE

_mesh = plsc.VectorSubcoreMesh(
    core_axis_name="c", subcore_axis_name="s", num_cores=NC, num_subcores=NS
)


@functools.partial(
    pl.kernel,
    out_type=(
        jax.ShapeDtypeStruct((N_NODES,), jnp.int32),
        jax.ShapeDtypeStruct((N_NODES,), jnp.int32),
    ),
    mesh=_mesh,
    scratch_types=[
        [pltpu.VMEM((STREAM,), jnp.int32) for _ in range(NSTREAM)],
        pltpu.VMEM((STREAM,), jnp.int32),
        pltpu.VMEM((SLICE,), jnp.int32),
        pltpu.VMEM_SHARED((N_NODES_PAD,), jnp.int32),
        pltpu.SemaphoreType.DMA,
        pltpu.SemaphoreType.DMA,
    ],
    compiler_params=pltpu.CompilerParams(use_tc_tiling_on_sc=False),
)
def _degree_sc(edge_hbm, out0_hbm, out1_hbm,
               idx_v, ones_v, zero_v, hist_s, sem, sem2):
    c = lax.axis_index("c")
    s = lax.axis_index("s")

    idx_cps = [
        pltpu.async_copy(
            edge_hbm.at[c, pl.ds(s * PER_SUBCORE + k * STREAM, STREAM)],
            idx_v[k],
            sem,
        )
        for k in range(NSTREAM)
    ]

    def fill_ones(i, carry):
        ones_v[pl.ds(pl.multiple_of(i * 16, 16), 16)] = jnp.full((16,), 1, jnp.int32)
        return carry

    lax.fori_loop(0, STREAM // 16, fill_ones, 0)
    for k in range(SLICE // 16):
        zero_v[pl.ds(k * 16, 16)] = jnp.zeros((16,), jnp.int32)

    pltpu.sync_copy(zero_v, hist_s.at[pl.ds(s * SLICE, SLICE)])
    plsc.subcore_barrier()

    adds = []
    for k in range(NSTREAM):
        idx_cps[k].wait()
        adds.append(
            pltpu.async_copy(ones_v, hist_s.at[idx_v[k]], sem2, add=True)
        )
    for cp in adds:
        cp.wait()
    plsc.subcore_barrier()

    @pl.when(s < NS - 1)
    def _full_slice():
        for dst in (out0_hbm, out1_hbm):
            @pl.when((c == 0) == (dst is out0_hbm))
            def _():
                pltpu.sync_copy(hist_s.at[pl.ds(s * SLICE, SLICE)],
                                dst.at[pl.ds(s * SLICE, SLICE)])

    @pl.when(s == NS - 1)
    def _tail_slice():
        for dst in (out0_hbm, out1_hbm):
            @pl.when((c == 0) == (dst is out0_hbm))
            def _():
                pltpu.sync_copy(hist_s.at[pl.ds((NS - 1) * SLICE, TAIL)],
                                dst.at[pl.ds((NS - 1) * SLICE, TAIL)])


def _copy_body(x_ref, o_ref):
    o_ref[...] = x_ref[...]


_copy_x = pl.pallas_call(
    _copy_body,
    grid=(5,),
    in_specs=[pl.BlockSpec((2000, 128), lambda i: (i, 0))],
    out_specs=pl.BlockSpec((2000, 128), lambda i: (i, 0)),
    out_shape=jax.ShapeDtypeStruct((10000, 128), jnp.float32),
)


def kernel(x, edge_index):
    ei = edge_index.astype(jnp.int32)
    out_degree, in_degree = _degree_sc(ei)
    x_out = _copy_x(x)
    out_dtype = jax.dtypes.canonicalize_dtype(jnp.int64)
    return x_out, in_degree.astype(out_dtype), out_degree.astype(out_dtype)
